# baseline (speedup 1.0000x reference)
import jax
import jax.numpy as jnp
from jax.experimental import pallas as pl
from jax.experimental.pallas import tpu as pltpu

_COPY_ROWS = 2048


def _copy_body(kc_ref, vc_ref, ko_ref, vo_ref):
    ko_ref[...] = kc_ref[...]
    vo_ref[...] = vc_ref[...]


def _scatter_body(pos_ref, kv_ref, vv_ref, k0_ref, v0_ref, ko_ref, vo_ref,
                  sem_ref):
    q = kv_ref.shape[1]
    copies = []
    for i in range(q):
        p = pos_ref[i]
        copies.append(pltpu.make_async_copy(
            kv_ref.at[:, pl.ds(i, 1), :], ko_ref.at[:, pl.ds(p, 1), :],
            sem_ref))
        copies.append(pltpu.make_async_copy(
            vv_ref.at[:, pl.ds(i, 1), :], vo_ref.at[:, pl.ds(p, 1), :],
            sem_ref))
    for c in copies:
        c.start()
    for c in copies:
        c.wait()


def kernel(input_pos, k_val, v_val, k_cache, v_cache):
    B, H, S, D = k_cache.shape
    Q = k_val.shape[2]
    BH = B * H
    rows = BH * S * D // 512
    kc = k_cache.reshape(rows, 512)
    vc = v_cache.reshape(rows, 512)

    k0, v0 = pl.pallas_call(
        _copy_body,
        grid=(rows // _COPY_ROWS,),
        in_specs=[
            pl.BlockSpec((_COPY_ROWS, 512), lambda i: (i, 0)),
            pl.BlockSpec((_COPY_ROWS, 512), lambda i: (i, 0)),
        ],
        out_specs=[
            pl.BlockSpec((_COPY_ROWS, 512), lambda i: (i, 0)),
            pl.BlockSpec((_COPY_ROWS, 512), lambda i: (i, 0)),
        ],
        out_shape=[jax.ShapeDtypeStruct((rows, 512), jnp.float32)] * 2,
        compiler_params=pltpu.CompilerParams(
            dimension_semantics=("arbitrary",)
        ),
    )(kc, vc)

    k0 = k0.reshape(BH, S, D)
    v0 = v0.reshape(BH, S, D)
    kv = k_val.reshape(BH, Q, D)
    vv = v_val.reshape(BH, Q, D)

    ko, vo = pl.pallas_call(
        _scatter_body,
        in_specs=[
            pl.BlockSpec(memory_space=pltpu.SMEM),
            pl.BlockSpec(memory_space=pltpu.VMEM),
            pl.BlockSpec(memory_space=pltpu.VMEM),
            pl.BlockSpec(memory_space=pl.ANY),
            pl.BlockSpec(memory_space=pl.ANY),
        ],
        out_specs=[
            pl.BlockSpec(memory_space=pl.ANY),
            pl.BlockSpec(memory_space=pl.ANY),
        ],
        out_shape=[jax.ShapeDtypeStruct((BH, S, D), jnp.float32)] * 2,
        input_output_aliases={3: 0, 4: 1},
        scratch_shapes=[pltpu.SemaphoreType.DMA],
    )(input_pos.astype(jnp.int32), kv, vv, k0, v0)

    return ko.reshape(B, H, S, D), vo.reshape(B, H, S, D)


# speedup vs baseline: 1.7906x; 1.7906x over previous
import jax
import jax.numpy as jnp
from jax.experimental import pallas as pl
from jax.experimental.pallas import tpu as pltpu

_BH_BLK = 4


def _copy_body(kc_ref, vc_ref, ko_ref, vo_ref):
    ko_ref[...] = kc_ref[...]
    vo_ref[...] = vc_ref[...]


def _scatter_body(pos_ref, kv_ref, vv_ref, k0_ref, v0_ref, ko_ref, vo_ref,
                  sem_ref):
    q = kv_ref.shape[1]
    copies = []
    for i in range(q):
        p = pos_ref[i]
        copies.append(pltpu.make_async_copy(
            kv_ref.at[:, pl.ds(i, 1), :], ko_ref.at[:, pl.ds(p, 1), :],
            sem_ref))
        copies.append(pltpu.make_async_copy(
            vv_ref.at[:, pl.ds(i, 1), :], vo_ref.at[:, pl.ds(p, 1), :],
            sem_ref))
    for c in copies:
        c.start()
    for c in copies:
        c.wait()


def kernel(input_pos, k_val, v_val, k_cache, v_cache):
    B, H, S, D = k_cache.shape
    Q = k_val.shape[2]
    BH = B * H
    kc = k_cache.reshape(BH, S, D)
    vc = v_cache.reshape(BH, S, D)

    k0, v0 = pl.pallas_call(
        _copy_body,
        grid=(BH // _BH_BLK,),
        in_specs=[
            pl.BlockSpec((_BH_BLK, S, D), lambda i: (i, 0, 0)),
            pl.BlockSpec((_BH_BLK, S, D), lambda i: (i, 0, 0)),
        ],
        out_specs=[
            pl.BlockSpec((_BH_BLK, S, D), lambda i: (i, 0, 0)),
            pl.BlockSpec((_BH_BLK, S, D), lambda i: (i, 0, 0)),
        ],
        out_shape=[jax.ShapeDtypeStruct((BH, S, D), jnp.float32)] * 2,
        compiler_params=pltpu.CompilerParams(
            dimension_semantics=("arbitrary",)
        ),
    )(kc, vc)

    kv = k_val.reshape(BH, Q, D)
    vv = v_val.reshape(BH, Q, D)

    ko, vo = pl.pallas_call(
        _scatter_body,
        in_specs=[
            pl.BlockSpec(memory_space=pltpu.SMEM),
            pl.BlockSpec(memory_space=pltpu.VMEM),
            pl.BlockSpec(memory_space=pltpu.VMEM),
            pl.BlockSpec(memory_space=pl.ANY),
            pl.BlockSpec(memory_space=pl.ANY),
        ],
        out_specs=[
            pl.BlockSpec(memory_space=pl.ANY),
            pl.BlockSpec(memory_space=pl.ANY),
        ],
        out_shape=[jax.ShapeDtypeStruct((BH, S, D), jnp.float32)] * 2,
        input_output_aliases={3: 0, 4: 1},
        scratch_shapes=[pltpu.SemaphoreType.DMA],
    )(input_pos.astype(jnp.int32), kv, vv, k0, v0)

    return ko.reshape(B, H, S, D), vo.reshape(B, H, S, D)
